# baseline (speedup 1.0000x reference)
import functools

import jax
import jax.numpy as jnp
from jax import lax
from jax.experimental import pallas as pl
from jax.experimental.pallas import tpu as pltpu
from jax.experimental.pallas import tpu_sc as plsc

_MAP_INDEX = 50000
_B = 4096
_NC = 2
_NS = 16
_NW = _NC * _NS
_PER_W = _B // _NW
_L = 16


def _sc_body(lt_hbm, out_hbm, x2_v, y_v, x_v, oidx0_v, oidx1_v, sem):
    wid = lax.axis_index("s") * _NC + lax.axis_index("c")
    base = wid * _PER_W
    lane = lax.iota(jnp.int32, _L)
    pltpu.sync_copy(lt_hbm.at[pl.ds(_MAP_INDEX, 1), pl.ds(base, _PER_W)], x2_v)
    for k in range(_PER_W // _L):
        sl = pl.ds(k * _L, _L)
        x = x2_v[0, sl]
        x_v[sl] = x
        y_v[sl] = 1.0 - x
        rows = base + (k * _L) + lane
        oidx0_v[sl] = rows * 2
        oidx1_v[sl] = rows * 2 + 1
    c0 = pltpu.async_copy(y_v, out_hbm.at[oidx0_v], sem)
    c1 = pltpu.async_copy(x_v, out_hbm.at[oidx1_v], sem)
    c0.wait()
    c1.wait()


@jax.jit
def kernel(logits):
    lt = logits.T
    run = functools.partial(
        pl.kernel,
        mesh=plsc.VectorSubcoreMesh(core_axis_name="c", subcore_axis_name="s"),
        out_type=jax.ShapeDtypeStruct((_B * 2,), jnp.float32),
        compiler_params=pltpu.CompilerParams(use_tc_tiling_on_sc=True),
        scratch_types=[
            pltpu.VMEM((1, _PER_W), jnp.float32),
            pltpu.VMEM((_PER_W,), jnp.float32),
            pltpu.VMEM((_PER_W,), jnp.float32),
            pltpu.VMEM((_PER_W,), jnp.int32),
            pltpu.VMEM((_PER_W,), jnp.int32),
            pltpu.SemaphoreType.DMA,
        ],
    )(_sc_body)
    return run(lt).reshape(_B, 2)


# speedup vs baseline: 47.4222x; 47.4222x over previous
import jax
import jax.numpy as jnp
from jax.experimental import pallas as pl
from jax.experimental.pallas import tpu as pltpu

_MAP_INDEX = 50000
_B = 4096


def _tc_body(hbm_ref, o_ref, x_vmem, sem):
    copy = pltpu.make_async_copy(
        hbm_ref.at[pl.ds(_MAP_INDEX, 1), :], x_vmem, sem
    )
    copy.start()
    copy.wait()
    x = x_vmem[0:1, :]
    o_ref[0:1, :] = 1.0 - x
    o_ref[1:2, :] = x


@jax.jit
def kernel(logits):
    lt = logits.T
    out = pl.pallas_call(
        _tc_body,
        in_specs=[pl.BlockSpec(memory_space=pl.ANY)],
        out_specs=pl.BlockSpec(memory_space=pltpu.VMEM),
        out_shape=jax.ShapeDtypeStruct((2, _B), logits.dtype),
        compiler_params=pltpu.CompilerParams(skip_device_barrier=True),
        scratch_shapes=[
            pltpu.VMEM((1, _B), jnp.float32),
            pltpu.SemaphoreType.DMA,
        ],
    )(lt)
    return out.T
